# baseline (speedup 1.0000x reference)
import jax
import jax.numpy as jnp
from jax.experimental import pallas as pl
from jax.experimental.pallas import tpu as pltpu

_B, _L, _D = 8, 512, 256
_P = 4
_KNN = 10
_SKIP = 0.8
_VERY_SMALL = 1e-12
_INF = 1e20
_SPP = 2


def _one_sample(i, ln, men_row, raw_ref, wenc_ref, benc_ref, wt_ref,
                w1_ref, b1_ref, w2_ref, b2_ref, w3_ref, b3_ref,
                out_ref, iadj_ref, radj_ref, cadj_ref, raw_out_ref, node_ref,
                h_ref):
    raw = raw_ref[i]

    colv = jax.lax.broadcasted_iota(jnp.int32, (1, _L), 1) < ln
    rowv = jax.lax.broadcasted_iota(jnp.int32, (_L, 1), 0) < ln
    mcol = colv.astype(jnp.float32)
    mrow = rowv.astype(jnp.float32)

    att = jax.lax.dot_general(raw, raw, (((1,), (1,)), ((), ())),
                              preferred_element_type=jnp.float32)
    x = jnp.where(colv & rowv, att, -_INF)

    t = jnp.max(x, axis=1, keepdims=True)
    for _ in range(_KNN - 1):
        t = jnp.max(jnp.where(x < t, x, -_INF), axis=1, keepdims=True)

    adj0 = ((x >= t) & colv & rowv).astype(jnp.float32)
    rs_col = jnp.sum(adj0, axis=1, keepdims=True)
    ones_row = jnp.ones((1, _L), jnp.float32)
    rs_row = jax.lax.dot_general(ones_row, adj0, (((1,), (1,)), ((), ())),
                                 preferred_element_type=jnp.float32)
    d_col = jax.lax.rsqrt(jnp.maximum(rs_col, _VERY_SMALL))
    d_row = jax.lax.rsqrt(jnp.maximum(rs_row, _VERY_SMALL))
    init_adj = adj0 * d_col * d_row

    wt = wt_ref[...]
    n2 = jax.lax.dot_general(raw * raw, wt * wt, (((1,), (1,)), ((), ())),
                             preferred_element_type=jnp.float32)
    inv_nrm = 1.0 / jnp.maximum(jnp.sqrt(n2), 1e-12)
    cfs = []
    for p in range(_P):
        cfs.append(raw * wt[p:p + 1, :] * inv_nrm[:, p:p + 1])
    cfhat = jnp.concatenate(cfs, axis=1)
    attg = jax.lax.dot_general(cfhat, cfhat, (((1,), (1,)), ((), ())),
                               preferred_element_type=jnp.float32)
    raw_adj = jnp.maximum(attg, 0.0) * ((mcol * (1.0 / _P)) * mrow)
    inv_rs = (1.0 - _SKIP) / jnp.maximum(
        jnp.sum(raw_adj, axis=1, keepdims=True), _VERY_SMALL)
    cur_adj = _SKIP * init_adj + raw_adj * inv_rs

    enc = jnp.tanh(
        jax.lax.dot_general(raw, wenc_ref[...], (((1,), (0,)), ((), ())),
                            preferred_element_type=jnp.float32) + benc_ref[...])
    s0, e0, s1, e1 = men_row[0], men_row[1], men_row[2], men_row[3]
    li = jax.lax.broadcasted_iota(jnp.int32, (1, _L), 1)
    sp0 = ((li >= s0) & (li <= e0)).astype(jnp.float32)
    sp1 = ((li >= s1) & (li <= e1)).astype(jnp.float32)
    arg1 = jax.lax.dot_general(sp0, enc, (((1,), (0,)), ((), ())),
                               preferred_element_type=jnp.float32)
    arg2 = jax.lax.dot_general(sp1, enc, (((1,), (0,)), ((), ())),
                               preferred_element_type=jnp.float32)
    arg1 = arg1 / (e0 - s0 + 1).astype(jnp.float32)
    arg2 = arg2 / (e1 - s1 + 1).astype(jnp.float32)
    node = enc + arg1 + arg2

    def mm(a, w):
        return jax.lax.dot_general(a, w, (((1,), (0,)), ((), ())),
                                   preferred_element_type=jnp.float32)

    h1 = jax.nn.relu(mm(cur_adj, mm(node, w1_ref[...])) + b1_ref[...])
    h2 = jax.nn.relu(mm(cur_adj, mm(h1, w2_ref[...])) + b2_ref[...])
    out = mm(cur_adj, mm(h2, w3_ref[...])) + b3_ref[...]

    out_ref[i] = out
    iadj_ref[i] = init_adj
    radj_ref[i] = raw_adj
    cadj_ref[i] = cur_adj
    raw_out_ref[i] = raw
    node_ref[i] = node
    h_ref[i] = h2


def _tg_kernel(len_ref, men_ref, raw_ref, *refs):
    g = pl.program_id(0)
    for i in range(_SPP):
        s = g * _SPP + i
        _one_sample(i, len_ref[s], (men_ref[s, 0], men_ref[s, 1],
                                    men_ref[s, 2], men_ref[s, 3]),
                    raw_ref, *refs)


def kernel(context_vec, context_len, mentions, W_enc, b_enc, weight_tensor,
           W1, b1, W2, b2, W3, b3):
    mask = (jnp.arange(_L)[None, :] < context_len[:, None]).astype(jnp.float32)

    def _c(shape):
        return pl.BlockSpec(shape, lambda b, *_: (0,) * len(shape))

    grid_spec = pltpu.PrefetchScalarGridSpec(
        num_scalar_prefetch=2,
        grid=(_B // _SPP,),
        in_specs=[
            pl.BlockSpec((_SPP, _L, _D), lambda b, *_: (b, 0, 0)),
            _c((_D, _D)), _c((1, _D)), _c((_P, _D)),
            _c((_D, _D)), _c((1, _D)),
            _c((_D, _D)), _c((1, _D)),
            _c((_D, _D)), _c((1, _D)),
        ],
        out_specs=[
            pl.BlockSpec((_SPP, _L, _D), lambda b, *_: (b, 0, 0)),
            pl.BlockSpec((_SPP, _L, _L), lambda b, *_: (b, 0, 0)),
            pl.BlockSpec((_SPP, _L, _L), lambda b, *_: (b, 0, 0)),
            pl.BlockSpec((_SPP, _L, _L), lambda b, *_: (b, 0, 0)),
            pl.BlockSpec((_SPP, _L, _D), lambda b, *_: (b, 0, 0)),
            pl.BlockSpec((_SPP, _L, _D), lambda b, *_: (b, 0, 0)),
            pl.BlockSpec((_SPP, _L, _D), lambda b, *_: (b, 0, 0)),
        ],
    )
    out_shapes = [
        jax.ShapeDtypeStruct((_B, _L, _D), jnp.float32),
        jax.ShapeDtypeStruct((_B, _L, _L), jnp.float32),
        jax.ShapeDtypeStruct((_B, _L, _L), jnp.float32),
        jax.ShapeDtypeStruct((_B, _L, _L), jnp.float32),
        jax.ShapeDtypeStruct((_B, _L, _D), jnp.float32),
        jax.ShapeDtypeStruct((_B, _L, _D), jnp.float32),
        jax.ShapeDtypeStruct((_B, _L, _D), jnp.float32),
    ]
    out, iadj, radj, cadj, raw_out, node, h = pl.pallas_call(
        _tg_kernel,
        grid_spec=grid_spec,
        out_shape=out_shapes,
        compiler_params=pltpu.CompilerParams(
            dimension_semantics=("arbitrary",)),
    )(context_len, mentions, context_vec, W_enc, b_enc.reshape(1, _D),
      weight_tensor, W1, b1.reshape(1, _D), W2, b2.reshape(1, _D),
      W3, b3.reshape(1, _D))
    return (out, (iadj, radj, cadj, raw_out, node, h, mask))


# speedup vs baseline: 1.1372x; 1.1372x over previous
import jax
import jax.numpy as jnp
from jax.experimental import pallas as pl
from jax.experimental.pallas import tpu as pltpu

_B, _L, _D = 8, 512, 256
_P = 4
_KNN = 10
_SKIP = 0.8
_VERY_SMALL = 1e-12
_INF = 1e20
_SPP = 2


def _mm(a, w):
    return jax.lax.dot_general(a, w, (((1,), (0,)), ((), ())),
                               preferred_element_type=jnp.float32)


def _mmt(a, b_):
    return jax.lax.dot_general(a, b_, (((1,), (1,)), ((), ())),
                               preferred_element_type=jnp.float32)


def _stage_att(raw, colv, rowv):
    att = _mmt(raw, raw)
    return jnp.where(colv & rowv, att, -_INF)


def _stage_topk(x):
    t = jnp.max(x, axis=1, keepdims=True)
    for _ in range(_KNN - 1):
        t = jnp.max(jnp.where(x < t, x, -_INF), axis=1, keepdims=True)
    return t


def _stage_init_adj(x, t, colv, rowv):
    adj0 = ((x >= t) & colv & rowv).astype(jnp.float32)
    rs_col = jnp.sum(adj0, axis=1, keepdims=True)
    ones_row = jnp.ones((1, _L), jnp.float32)
    rs_row = _mmt(ones_row, adj0)
    d_col = jax.lax.rsqrt(jnp.maximum(rs_col, _VERY_SMALL))
    d_row = jax.lax.rsqrt(jnp.maximum(rs_row, _VERY_SMALL))
    return adj0 * d_col * d_row


def _stage_learner_mm(raw, wt):
    n2 = _mmt(raw * raw, wt * wt)
    inv_nrm = 1.0 / jnp.maximum(jnp.sqrt(n2), 1e-12)
    cfs = []
    for p in range(_P):
        cfs.append(raw * wt[p:p + 1, :] * inv_nrm[:, p:p + 1])
    cfhat = jnp.concatenate(cfs, axis=1)
    return _mmt(cfhat, cfhat)


def _stage_adj_mix(attg, init_adj, mcol, mrow):
    raw_adj = jnp.maximum(attg, 0.0) * ((mcol * (1.0 / _P)) * mrow)
    inv_rs = (1.0 - _SKIP) / jnp.maximum(
        jnp.sum(raw_adj, axis=1, keepdims=True), _VERY_SMALL)
    cur_adj = _SKIP * init_adj + raw_adj * inv_rs
    return raw_adj, cur_adj


def _stage_node(raw, men_row, wenc, benc):
    enc = jnp.tanh(_mm(raw, wenc) + benc)
    s0, e0, s1, e1 = men_row
    li = jax.lax.broadcasted_iota(jnp.int32, (1, _L), 1)
    sp0 = ((li >= s0) & (li <= e0)).astype(jnp.float32)
    sp1 = ((li >= s1) & (li <= e1)).astype(jnp.float32)
    arg1 = _mm(sp0, enc) / (e0 - s0 + 1).astype(jnp.float32)
    arg2 = _mm(sp1, enc) / (e1 - s1 + 1).astype(jnp.float32)
    return enc + arg1 + arg2


def _stage_gcn(cur_adj, node, w1, b1, w2, b2, w3, b3):
    h1 = jax.nn.relu(_mm(cur_adj, _mm(node, w1)) + b1)
    h2 = jax.nn.relu(_mm(cur_adj, _mm(h1, w2)) + b2)
    out = _mm(cur_adj, _mm(h2, w3)) + b3
    return h2, out


def _tg_kernel(len_ref, men_ref, raw_ref, wenc_ref, benc_ref, wt_ref,
               w1_ref, b1_ref, w2_ref, b2_ref, w3_ref, b3_ref,
               out_ref, iadj_ref, radj_ref, cadj_ref, raw_out_ref, node_ref,
               h_ref):
    g = pl.program_id(0)
    wt = wt_ref[...]

    raw = [raw_ref[i] for i in range(_SPP)]
    ln = [len_ref[g * _SPP + i] for i in range(_SPP)]
    men = [tuple(men_ref[g * _SPP + i, j] for j in range(4))
           for i in range(_SPP)]
    colv = [jax.lax.broadcasted_iota(jnp.int32, (1, _L), 1) < ln[i]
            for i in range(_SPP)]
    rowv = [jax.lax.broadcasted_iota(jnp.int32, (_L, 1), 0) < ln[i]
            for i in range(_SPP)]
    mcol = [c.astype(jnp.float32) for c in colv]
    mrow = [r.astype(jnp.float32) for r in rowv]

    x0 = _stage_att(raw[0], colv[0], rowv[0])
    x1 = _stage_att(raw[1], colv[1], rowv[1])

    t0 = _stage_topk(x0)
    attg1 = _stage_learner_mm(raw[1], wt)

    t1 = _stage_topk(x1)
    attg0 = _stage_learner_mm(raw[0], wt)
    node0 = _stage_node(raw[0], men[0], wenc_ref[...], benc_ref[...])
    node1 = _stage_node(raw[1], men[1], wenc_ref[...], benc_ref[...])

    iadj0 = _stage_init_adj(x0, t0, colv[0], rowv[0])
    iadj1 = _stage_init_adj(x1, t1, colv[1], rowv[1])
    radj0, cadj0 = _stage_adj_mix(attg0, iadj0, mcol[0], mrow[0])
    radj1, cadj1 = _stage_adj_mix(attg1, iadj1, mcol[1], mrow[1])

    h0, out0 = _stage_gcn(cadj0, node0, w1_ref[...], b1_ref[...],
                          w2_ref[...], b2_ref[...], w3_ref[...], b3_ref[...])
    h1, out1 = _stage_gcn(cadj1, node1, w1_ref[...], b1_ref[...],
                          w2_ref[...], b2_ref[...], w3_ref[...], b3_ref[...])

    for i, (o, ia, ra, ca, nd, hh) in enumerate(
            [(out0, iadj0, radj0, cadj0, node0, h0),
             (out1, iadj1, radj1, cadj1, node1, h1)]):
        out_ref[i] = o
        iadj_ref[i] = ia
        radj_ref[i] = ra
        cadj_ref[i] = ca
        raw_out_ref[i] = raw[i]
        node_ref[i] = nd
        h_ref[i] = hh


def kernel(context_vec, context_len, mentions, W_enc, b_enc, weight_tensor,
           W1, b1, W2, b2, W3, b3):
    mask = (jnp.arange(_L)[None, :] < context_len[:, None]).astype(jnp.float32)

    def _c(shape):
        return pl.BlockSpec(shape, lambda b, *_: (0,) * len(shape))

    grid_spec = pltpu.PrefetchScalarGridSpec(
        num_scalar_prefetch=2,
        grid=(_B // _SPP,),
        in_specs=[
            pl.BlockSpec((_SPP, _L, _D), lambda b, *_: (b, 0, 0)),
            _c((_D, _D)), _c((1, _D)), _c((_P, _D)),
            _c((_D, _D)), _c((1, _D)),
            _c((_D, _D)), _c((1, _D)),
            _c((_D, _D)), _c((1, _D)),
        ],
        out_specs=[
            pl.BlockSpec((_SPP, _L, _D), lambda b, *_: (b, 0, 0)),
            pl.BlockSpec((_SPP, _L, _L), lambda b, *_: (b, 0, 0)),
            pl.BlockSpec((_SPP, _L, _L), lambda b, *_: (b, 0, 0)),
            pl.BlockSpec((_SPP, _L, _L), lambda b, *_: (b, 0, 0)),
            pl.BlockSpec((_SPP, _L, _D), lambda b, *_: (b, 0, 0)),
            pl.BlockSpec((_SPP, _L, _D), lambda b, *_: (b, 0, 0)),
            pl.BlockSpec((_SPP, _L, _D), lambda b, *_: (b, 0, 0)),
        ],
    )
    out_shapes = [
        jax.ShapeDtypeStruct((_B, _L, _D), jnp.float32),
        jax.ShapeDtypeStruct((_B, _L, _L), jnp.float32),
        jax.ShapeDtypeStruct((_B, _L, _L), jnp.float32),
        jax.ShapeDtypeStruct((_B, _L, _L), jnp.float32),
        jax.ShapeDtypeStruct((_B, _L, _D), jnp.float32),
        jax.ShapeDtypeStruct((_B, _L, _D), jnp.float32),
        jax.ShapeDtypeStruct((_B, _L, _D), jnp.float32),
    ]
    out, iadj, radj, cadj, raw_out, node, h = pl.pallas_call(
        _tg_kernel,
        grid_spec=grid_spec,
        out_shape=out_shapes,
        compiler_params=pltpu.CompilerParams(
            dimension_semantics=("arbitrary",)),
    )(context_len, mentions, context_vec, W_enc, b_enc.reshape(1, _D),
      weight_tensor, W1, b1.reshape(1, _D), W2, b2.reshape(1, _D),
      W3, b3.reshape(1, _D))
    return (out, (iadj, radj, cadj, raw_out, node, h, mask))
